# baseline (speedup 1.0000x reference)
import jax
import jax.numpy as jnp
from jax.experimental import pallas as pl
from jax.experimental.pallas import tpu as pltpu

_SUBLANE = 8
_LANE = 128
_GROUP = 16
_PACK = _LANE // _GROUP
_TILE_THRESHOLD = 1024


def _round_up(n, m):
    return ((n + m - 1) // m) * m


def _body_packed(in_dim):
    def body(x_ref, w_ref, o_ref):
        x = x_ref[...]
        w = w_ref[...]
        for u in range(_PACK):
            o_ref[:, u, :] = jax.lax.dot_general(
                x[:, u * _GROUP:(u + 1) * _GROUP],
                w,
                dimension_numbers=(((1,), (0,)), ((), ())),
                preferred_element_type=jnp.float32,
            ).astype(o_ref.dtype)

    return body


def _forward_packed(xd, w16, tile_d, in_dim):
    dense_rows = xd.shape[0]
    grid = (dense_rows // tile_d,)
    out3 = pl.pallas_call(
        _body_packed(in_dim),
        out_shape=jax.ShapeDtypeStruct((dense_rows, _PACK, _LANE), xd.dtype),
        grid=grid,
        in_specs=[
            pl.BlockSpec((tile_d, _LANE), lambda i: (i, 0)),
            pl.BlockSpec((_GROUP, _LANE), lambda i: (0, 0)),
        ],
        out_specs=pl.BlockSpec((tile_d, _PACK, _LANE), lambda i: (i, 0, 0)),
        compiler_params=pltpu.CompilerParams(
            dimension_semantics=("parallel",)
        ),
        cost_estimate=pl.CostEstimate(
            flops=2 * dense_rows * _PACK * in_dim * _LANE,
            transcendentals=0,
            bytes_accessed=4 * (dense_rows * _LANE
                                + dense_rows * _PACK * _LANE),
        ),
    )(xd, w16)
    return out3.reshape(dense_rows * _PACK, _LANE)


def _body2d(in_dim):
    def body(x_ref, w_ref, o_ref):
        o_ref[...] = jax.lax.dot_general(
            x_ref[...],
            w_ref[0:in_dim, :],
            dimension_numbers=(((1,), (0,)), ((), ())),
            preferred_element_type=jnp.float32,
        ).astype(o_ref.dtype)

    return body


def _forward2d(x, wt_pad, rows_pad, tile_rows):
    rows, in_dim = x.shape
    in_pad, out_pad = wt_pad.shape
    if rows_pad != rows:
        x = jnp.pad(x, ((0, rows_pad - rows), (0, 0)))
    grid = (rows_pad // tile_rows,)
    return pl.pallas_call(
        _body2d(in_dim),
        out_shape=jax.ShapeDtypeStruct((rows_pad, out_pad), x.dtype),
        grid=grid,
        in_specs=[
            pl.BlockSpec((tile_rows, in_dim), lambda i: (i, 0)),
            pl.BlockSpec((in_pad, out_pad), lambda i: (0, 0)),
        ],
        out_specs=pl.BlockSpec((tile_rows, out_pad), lambda i: (i, 0)),
        compiler_params=pltpu.CompilerParams(
            dimension_semantics=("parallel",)
        ),
    )(x, wt_pad)


@jax.jit
def kernel(xs, wt_pad):
    n, b, in_dim = xs.shape
    rows = n * b
    if rows < _TILE_THRESHOLD:
        rows_pad = _round_up(max(rows, _SUBLANE), _SUBLANE)
        return _forward2d(xs.reshape(rows, in_dim), wt_pad, rows_pad,
                          rows_pad)
    if rows % 512 == 0 and in_dim <= _GROUP:
        xpad = jnp.pad(xs.reshape(rows, in_dim),
                       ((0, 0), (0, _GROUP - in_dim)))
        xd = xpad.reshape(rows // _PACK, _LANE)
        w16 = wt_pad[0:_GROUP, :]
        dense_rows = rows // _PACK
        tile_d = 64
        for cand in (1024, 512, 256, 128):
            if dense_rows % cand == 0:
                tile_d = cand
                break
        return _forward_packed(xd, w16, tile_d, in_dim)
    rows_pad = _round_up(rows, 512)
    return _forward2d(xs.reshape(rows, in_dim), wt_pad, rows_pad, 512)


# speedup vs baseline: 2.3916x; 2.3916x over previous
import jax
import jax.numpy as jnp
from jax.experimental import pallas as pl
from jax.experimental.pallas import tpu as pltpu

_SUBLANE = 8
_LANE = 128
_GROUP = 16
_TILE_THRESHOLD = 1024


def _round_up(n, m):
    return ((n + m - 1) // m) * m


def _body_t(in_dim):
    def body(x_ref, w_ref, o_ref):
        o_ref[...] = jax.lax.dot_general(
            x_ref[...],
            w_ref[...],
            dimension_numbers=(((0,), (0,)), ((), ())),
            preferred_element_type=jnp.float32,
        ).astype(o_ref.dtype)

    return body


def _forward_t(xt, w16, tile_r, in_dim):
    rows = xt.shape[1]
    grid = (rows // tile_r,)
    return pl.pallas_call(
        _body_t(in_dim),
        out_shape=jax.ShapeDtypeStruct((rows, _LANE), xt.dtype),
        grid=grid,
        in_specs=[
            pl.BlockSpec((_GROUP, tile_r), lambda i: (0, i)),
            pl.BlockSpec((_GROUP, _LANE), lambda i: (0, 0)),
        ],
        out_specs=pl.BlockSpec((tile_r, _LANE), lambda i: (i, 0)),
        compiler_params=pltpu.CompilerParams(
            dimension_semantics=("parallel",)
        ),
        cost_estimate=pl.CostEstimate(
            flops=2 * rows * in_dim * _LANE,
            transcendentals=0,
            bytes_accessed=4 * (rows * _GROUP + rows * _LANE),
        ),
    )(xt, w16)


def _body2d(in_dim):
    def body(x_ref, w_ref, o_ref):
        o_ref[...] = jax.lax.dot_general(
            x_ref[...],
            w_ref[0:in_dim, :],
            dimension_numbers=(((1,), (0,)), ((), ())),
            preferred_element_type=jnp.float32,
        ).astype(o_ref.dtype)

    return body


def _forward2d(x, wt_pad, rows_pad, tile_rows):
    rows, in_dim = x.shape
    in_pad, out_pad = wt_pad.shape
    if rows_pad != rows:
        x = jnp.pad(x, ((0, rows_pad - rows), (0, 0)))
    grid = (rows_pad // tile_rows,)
    return pl.pallas_call(
        _body2d(in_dim),
        out_shape=jax.ShapeDtypeStruct((rows_pad, out_pad), x.dtype),
        grid=grid,
        in_specs=[
            pl.BlockSpec((tile_rows, in_dim), lambda i: (i, 0)),
            pl.BlockSpec((in_pad, out_pad), lambda i: (0, 0)),
        ],
        out_specs=pl.BlockSpec((tile_rows, out_pad), lambda i: (i, 0)),
        compiler_params=pltpu.CompilerParams(
            dimension_semantics=("parallel",)
        ),
    )(x, wt_pad)


@jax.jit
def kernel(xs, wt_pad):
    n, b, in_dim = xs.shape
    rows = n * b
    if rows < _TILE_THRESHOLD:
        rows_pad = _round_up(max(rows, _SUBLANE), _SUBLANE)
        return _forward2d(xs.reshape(rows, in_dim), wt_pad, rows_pad,
                          rows_pad)
    if rows % 512 == 0 and in_dim <= _GROUP:
        x2 = xs.reshape(rows, in_dim)
        xt = jnp.pad(x2.T, ((0, _GROUP - in_dim), (0, 0)))
        w16 = wt_pad[0:_GROUP, :]
        tile_r = 512
        for cand in (4096, 2048, 1024):
            if rows % cand == 0:
                tile_r = cand
                break
        return _forward_t(xt, w16, tile_r, in_dim)
    rows_pad = _round_up(rows, 512)
    return _forward2d(xs.reshape(rows, in_dim), wt_pad, rows_pad, 512)
